# baseline (speedup 1.0000x reference)
import jax
import jax.numpy as jnp
from jax.experimental import pallas as pl
from jax.experimental.pallas import tpu as pltpu

_LANES = 128
_WROWS = 8
_WIN = _WROWS * _LANES


def _fixup_kernel(idx_ref, a_ref, o_ref):
    i = pl.program_id(0)
    idx = idx_ref[0]
    wb = (idx + i) // _WIN
    base = wb * _WIN
    rows = jax.lax.broadcasted_iota(jnp.int32, (_WROWS, _LANES), 0)
    cols = jax.lax.broadcasted_iota(jnp.int32, (_WROWS, _LANES), 1)
    flat = base + rows * _LANES + cols
    mask = jnp.logical_or(flat == idx, flat == idx + 1)
    o_ref[...] = jnp.where(mask, jnp.float32(0), a_ref[...])


def kernel(a, idx):
    n = a.shape[0]
    rows = n // _LANES
    idx32 = idx.astype(jnp.int32)
    a2 = a.reshape(rows, _LANES)

    def _imap(i, idx_ref):
        return ((idx_ref[0] + i) // _WIN, 0)

    out = pl.pallas_call(
        _fixup_kernel,
        out_shape=jax.ShapeDtypeStruct((rows, _LANES), a.dtype),
        grid_spec=pltpu.PrefetchScalarGridSpec(
            num_scalar_prefetch=1,
            grid=(2,),
            in_specs=[pl.BlockSpec((_WROWS, _LANES), _imap)],
            out_specs=pl.BlockSpec((_WROWS, _LANES), _imap),
        ),
        input_output_aliases={1: 0},
    )(idx32, a2)
    return out.reshape(n)


# speedup vs baseline: 1.0333x; 1.0333x over previous
import jax
import jax.numpy as jnp
from jax.experimental import pallas as pl
from jax.experimental.pallas import tpu as pltpu

_LANES = 128
_BLOCK_ROWS = 16384
_BLOCK = _BLOCK_ROWS * _LANES


def _copy_zero_kernel(idx_ref, a_ref, o_ref):
    i = pl.program_id(0)
    idx = idx_ref[0]
    base = i * _BLOCK

    contains = jnp.logical_and(idx + 1 >= base, idx < base + _BLOCK)

    @pl.when(jnp.logical_not(contains))
    def _plain():
        o_ref[...] = a_ref[...]

    @pl.when(contains)
    def _masked():
        rows = jax.lax.broadcasted_iota(jnp.int32, (_BLOCK_ROWS, _LANES), 0)
        cols = jax.lax.broadcasted_iota(jnp.int32, (_BLOCK_ROWS, _LANES), 1)
        flat = base + rows * _LANES + cols
        mask = jnp.logical_or(flat == idx, flat == idx + 1)
        o_ref[...] = jnp.where(mask, jnp.float32(0), a_ref[...])


def kernel(a, idx):
    n = a.shape[0]
    rows = n // _LANES
    grid = rows // _BLOCK_ROWS
    idx32 = idx.astype(jnp.int32)
    a2 = a.reshape(rows, _LANES)
    out = pl.pallas_call(
        _copy_zero_kernel,
        out_shape=jax.ShapeDtypeStruct((rows, _LANES), a.dtype),
        grid=(grid,),
        in_specs=[
            pl.BlockSpec(memory_space=pltpu.SMEM),
            pl.BlockSpec((_BLOCK_ROWS, _LANES), lambda i: (i, 0)),
        ],
        out_specs=pl.BlockSpec((_BLOCK_ROWS, _LANES), lambda i: (i, 0)),
    )(idx32, a2)
    return out.reshape(n)
